# baseline (speedup 1.0000x reference)
import functools

import jax
import jax.numpy as jnp
from jax import lax
from jax.experimental import pallas as pl
from jax.experimental.pallas import tpu as pltpu
from jax.experimental.pallas import tpu_sc as plsc

VOCAB = 100000
EMBED = 128
SEQ_LEN = 200
BATCH = 1024

NC, NS = 2, 16
NW = NC * NS
ROWS = BATCH * SEQ_LEN
C = 100
CHUNKS = ROWS // C
SEQ_PER_W = BATCH // NW
LANES = 16
COEF = 1.0 / (EMBED ** 0.5)

_mesh = plsc.VectorSubcoreMesh(
    core_axis_name="c", subcore_axis_name="s", num_cores=NC, num_subcores=NS
)


@functools.partial(
    pl.kernel,
    out_type=jax.ShapeDtypeStruct((BATCH, SEQ_LEN, EMBED), jnp.float32),
    mesh=_mesh,
    scratch_types=[
        pltpu.VMEM((CHUNKS // NW, C), jnp.int32),
        pltpu.VMEM((SEQ_LEN, EMBED // 2), jnp.int32),
        pltpu.VMEM((2, C, EMBED), jnp.float32),
        pltpu.VMEM((2, SEQ_LEN, EMBED), jnp.float32),
        pltpu.SemaphoreType.DMA,
        pltpu.SemaphoreType.DMA,
        pltpu.SemaphoreType.DMA,
        pltpu.SemaphoreType.DMA,
    ],
)
def _emb_lookup(x_ref, table_ref, pos_ref, out_ref,
                idx_v, pos_v, gbuf, obuf, gsem0, gsem1, ssem0, ssem1):
    wid = lax.axis_index("s") * NC + lax.axis_index("c")
    gsems = (gsem0, gsem1)
    ssems = (ssem0, ssem1)

    pltpu.sync_copy(x_ref.at[pl.ds(wid * (CHUNKS // NW), CHUNKS // NW)], idx_v)

    def gather(s, b):
        return pltpu.make_async_copy(
            table_ref.at[idx_v.at[2 * s + b]], gbuf.at[b], gsems[b])

    def store(s, o):
        return pltpu.make_async_copy(
            obuf.at[o], out_ref.at[wid * SEQ_PER_W + s], ssems[o])

    gather(0, 0).start()
    gather(0, 1).start()

    pltpu.sync_copy(pos_ref, obuf.at[0])

    def pack_row(r, _):
        def rne16(v):
            vi = lax.bitcast_convert_type(v, jnp.int32)
            return lax.shift_right_arithmetic(
                vi + jnp.int32(0x7FFF)
                + lax.bitwise_and(
                    lax.shift_right_arithmetic(vi, 16), jnp.int32(1)),
                16)

        src = obuf.at[0]
        for q in range(EMBED // (2 * LANES)):
            lo = lax.bitwise_and(
                rne16(src[r, pl.ds(q * 2 * LANES, LANES)]), jnp.int32(0xFFFF))
            hi = lax.shift_left(
                rne16(src[r, pl.ds(q * 2 * LANES + LANES, LANES)]), 16)
            pos_v[r, pl.ds(q * LANES, LANES)] = lax.bitwise_or(lo, hi)
        return 0

    lax.fori_loop(0, SEQ_LEN, pack_row, 0)

    def outer(oi, carry):
        for o in range(2):
            s = oi * 2 + o

            @pl.when(s >= 2)
            def _():
                store(s - 2, o).wait()

            ob = obuf.at[o]
            for b in range(2):
                gather(s, b).wait()
                gb = gbuf.at[b]

                @plsc.parallel_loop(0, C)
                def row(r):
                    rb = b * C + r
                    for q in range(EMBED // (2 * LANES)):
                        w = pos_v[rb, pl.ds(q * LANES, LANES)]
                        lo = lax.bitcast_convert_type(
                            lax.shift_left(w, 16), jnp.float32)
                        hi = lax.bitcast_convert_type(
                            lax.bitwise_and(w, jnp.int32(-65536)), jnp.float32)
                        sl0 = pl.ds(q * 2 * LANES, LANES)
                        sl1 = pl.ds(q * 2 * LANES + LANES, LANES)
                        ob[rb, sl0] = gb[r, sl0] * COEF + lo
                        ob[rb, sl1] = gb[r, sl1] * COEF + hi

                @pl.when(s < SEQ_PER_W - 1)
                def _():
                    gather(s + 1, b).start()

            store(s, o).start()
        return carry

    lax.fori_loop(0, SEQ_PER_W // 2, outer, 0)
    store(SEQ_PER_W - 2, 0).wait()
    store(SEQ_PER_W - 1, 1).wait()


def kernel(x, table, pos_enc):
    xf = x.astype(jnp.int32).reshape(CHUNKS, C)
    return _emb_lookup(xf, table, pos_enc)


# speedup vs baseline: 1.0008x; 1.0008x over previous
import functools

import jax
import jax.numpy as jnp
from jax import lax
from jax.experimental import pallas as pl
from jax.experimental.pallas import tpu as pltpu
from jax.experimental.pallas import tpu_sc as plsc

VOCAB = 100000
EMBED = 128
SEQ_LEN = 200
BATCH = 1024

NC, NS = 2, 16
NW = NC * NS
ROWS = BATCH * SEQ_LEN
C = 100
CHUNKS = ROWS // C
SEQ_PER_W = BATCH // NW
LANES = 16
COEF = 1.0 / (EMBED ** 0.5)

_mesh = plsc.VectorSubcoreMesh(
    core_axis_name="c", subcore_axis_name="s", num_cores=NC, num_subcores=NS
)


@functools.partial(
    pl.kernel,
    out_type=jax.ShapeDtypeStruct((BATCH, SEQ_LEN, EMBED), jnp.float32),
    mesh=_mesh,
    scratch_types=[
        pltpu.VMEM((CHUNKS // NW, C), jnp.int32),
        pltpu.VMEM((SEQ_LEN, EMBED // 2), jnp.int32),
        pltpu.VMEM((2, C, EMBED), jnp.float32),
        pltpu.VMEM((2, SEQ_LEN, EMBED), jnp.float32),
        pltpu.SemaphoreType.DMA,
        pltpu.SemaphoreType.DMA,
        pltpu.SemaphoreType.DMA,
        pltpu.SemaphoreType.DMA,
    ],
)
def _emb_lookup(x_ref, table_ref, pos_ref, out_ref,
                idx_v, pos_v, gbuf, obuf, gsem0, gsem1, ssem0, ssem1):
    wid = lax.axis_index("s") * NC + lax.axis_index("c")
    gsems = (gsem0, gsem1)
    ssems = (ssem0, ssem1)

    pltpu.sync_copy(x_ref.at[pl.ds(wid * (CHUNKS // NW), CHUNKS // NW)], idx_v)

    def gather(s, b):
        return pltpu.make_async_copy(
            table_ref.at[idx_v.at[2 * s + b]], gbuf.at[b], gsems[b])

    def store(s, o):
        return pltpu.make_async_copy(
            obuf.at[o], out_ref.at[wid * SEQ_PER_W + s], ssems[o])

    gather(0, 0).start()
    gather(0, 1).start()

    pltpu.sync_copy(pos_ref, obuf.at[0])

    @plsc.parallel_loop(0, SEQ_LEN)
    def pack_row(r):
        def rne16(v):
            vi = lax.bitcast_convert_type(v, jnp.int32)
            return lax.shift_right_arithmetic(
                vi + jnp.int32(0x7FFF)
                + lax.bitwise_and(
                    lax.shift_right_arithmetic(vi, 16), jnp.int32(1)),
                16)

        src = obuf.at[0]
        for q in range(EMBED // (2 * LANES)):
            lo = lax.bitwise_and(
                rne16(src[r, pl.ds(q * 2 * LANES, LANES)]), jnp.int32(0xFFFF))
            hi = lax.shift_left(
                rne16(src[r, pl.ds(q * 2 * LANES + LANES, LANES)]), 16)
            pos_v[r, pl.ds(q * LANES, LANES)] = lax.bitwise_or(lo, hi)

    def outer(oi, carry):
        for o in range(2):
            s = oi * 2 + o

            @pl.when(s >= 2)
            def _():
                store(s - 2, o).wait()

            ob = obuf.at[o]
            for b in range(2):
                gather(s, b).wait()
                gb = gbuf.at[b]

                @plsc.parallel_loop(0, C)
                def row(r):
                    rb = b * C + r
                    for q in range(EMBED // (2 * LANES)):
                        w = pos_v[rb, pl.ds(q * LANES, LANES)]
                        lo = lax.bitcast_convert_type(
                            lax.shift_left(w, 16), jnp.float32)
                        hi = lax.bitcast_convert_type(
                            lax.bitwise_and(w, jnp.int32(-65536)), jnp.float32)
                        sl0 = pl.ds(q * 2 * LANES, LANES)
                        sl1 = pl.ds(q * 2 * LANES + LANES, LANES)
                        ob[rb, sl0] = gb[r, sl0] * COEF + lo
                        ob[rb, sl1] = gb[r, sl1] * COEF + hi

                @pl.when(s < SEQ_PER_W - 1)
                def _():
                    gather(s + 1, b).start()

            store(s, o).start()
        return carry

    lax.fori_loop(0, SEQ_PER_W // 2, outer, 0)
    store(SEQ_PER_W - 2, 0).wait()
    store(SEQ_PER_W - 1, 1).wait()


def kernel(x, table, pos_enc):
    xf = x.astype(jnp.int32).reshape(CHUNKS, C)
    return _emb_lookup(xf, table, pos_enc)


# speedup vs baseline: 1.0491x; 1.0483x over previous
import functools

import jax
import jax.numpy as jnp
from jax import lax
from jax.experimental import pallas as pl
from jax.experimental.pallas import tpu as pltpu
from jax.experimental.pallas import tpu_sc as plsc

VOCAB = 100000
EMBED = 128
SEQ_LEN = 200
BATCH = 1024

NC, NS = 2, 16
NW = NC * NS
ROWS = BATCH * SEQ_LEN
C = 50
CHUNKS = ROWS // C
SEQ_PER_W = BATCH // NW
LANES = 16
COEF = 1.0 / (EMBED ** 0.5)

_mesh = plsc.VectorSubcoreMesh(
    core_axis_name="c", subcore_axis_name="s", num_cores=NC, num_subcores=NS
)


@functools.partial(
    pl.kernel,
    out_type=jax.ShapeDtypeStruct((BATCH, SEQ_LEN, EMBED), jnp.float32),
    mesh=_mesh,
    scratch_types=[
        pltpu.VMEM((CHUNKS // NW, C), jnp.int32),
        pltpu.VMEM((SEQ_LEN, EMBED // 2), jnp.int32),
        pltpu.VMEM((4, C, EMBED), jnp.float32),
        pltpu.VMEM((2, SEQ_LEN, EMBED), jnp.float32),
        pltpu.SemaphoreType.DMA,
        pltpu.SemaphoreType.DMA,
        pltpu.SemaphoreType.DMA,
        pltpu.SemaphoreType.DMA,
        pltpu.SemaphoreType.DMA,
        pltpu.SemaphoreType.DMA,
    ],
)
def _emb_lookup(x_ref, table_ref, pos_ref, out_ref,
                idx_v, pos_v, gbuf, obuf,
                gsem0, gsem1, gsem2, gsem3, ssem0, ssem1):
    wid = lax.axis_index("s") * NC + lax.axis_index("c")
    gsems = (gsem0, gsem1, gsem2, gsem3)
    ssems = (ssem0, ssem1)

    pltpu.sync_copy(x_ref.at[pl.ds(wid * (CHUNKS // NW), CHUNKS // NW)], idx_v)

    def gather(s, b):
        return pltpu.make_async_copy(
            table_ref.at[idx_v.at[4 * s + b]], gbuf.at[b], gsems[b])

    def store(s, o):
        return pltpu.make_async_copy(
            obuf.at[o], out_ref.at[wid * SEQ_PER_W + s], ssems[o])

    for b0 in range(4):
        gather(0, b0).start()

    pltpu.sync_copy(pos_ref, obuf.at[0])

    @plsc.parallel_loop(0, SEQ_LEN)
    def pack_row(r):
        def rne16(v):
            vi = lax.bitcast_convert_type(v, jnp.int32)
            return lax.shift_right_arithmetic(
                vi + jnp.int32(0x7FFF)
                + lax.bitwise_and(
                    lax.shift_right_arithmetic(vi, 16), jnp.int32(1)),
                16)

        src = obuf.at[0]
        for q in range(EMBED // (2 * LANES)):
            lo = lax.bitwise_and(
                rne16(src[r, pl.ds(q * 2 * LANES, LANES)]), jnp.int32(0xFFFF))
            hi = lax.shift_left(
                rne16(src[r, pl.ds(q * 2 * LANES + LANES, LANES)]), 16)
            pos_v[r, pl.ds(q * LANES, LANES)] = lax.bitwise_or(lo, hi)

    def outer(oi, carry):
        for o in range(2):
            s = oi * 2 + o

            @pl.when(s >= 2)
            def _():
                store(s - 2, o).wait()

            ob = obuf.at[o]
            for b in range(4):
                gather(s, b).wait()
                gb = gbuf.at[b]

                @plsc.parallel_loop(0, C)
                def row(r):
                    rb = b * C + r
                    for q in range(EMBED // (2 * LANES)):
                        w = pos_v[rb, pl.ds(q * LANES, LANES)]
                        lo = lax.bitcast_convert_type(
                            lax.shift_left(w, 16), jnp.float32)
                        hi = lax.bitcast_convert_type(
                            lax.bitwise_and(w, jnp.int32(-65536)), jnp.float32)
                        sl0 = pl.ds(q * 2 * LANES, LANES)
                        sl1 = pl.ds(q * 2 * LANES + LANES, LANES)
                        ob[rb, sl0] = gb[r, sl0] * COEF + lo
                        ob[rb, sl1] = gb[r, sl1] * COEF + hi

                @pl.when(s < SEQ_PER_W - 1)
                def _():
                    gather(s + 1, b).start()

            store(s, o).start()
        return carry

    lax.fori_loop(0, SEQ_PER_W // 2, outer, 0)
    store(SEQ_PER_W - 2, 0).wait()
    store(SEQ_PER_W - 1, 1).wait()


def kernel(x, table, pos_enc):
    xf = x.astype(jnp.int32).reshape(CHUNKS, C)
    return _emb_lookup(xf, table, pos_enc)
